# baseline (speedup 1.0000x reference)
import jax
import jax.numpy as jnp
from jax import lax
from jax.experimental import pallas as pl
from jax.experimental.pallas import tpu as pltpu
from jax.experimental.pallas import tpu_sc as plsc

NC, NS, L = 2, 16, 16
SLOW_CORE = 1
F = 48
CH = 128
R = 512


def _round_up(a, m):
    return (a + m - 1) // m * m


def _sc_degree(dst_rows, n_pad):
    epr = dst_rows.shape[0]
    cpt = epr // (NC * NS)
    npt = n_pad // NS

    def body(dst_hbm, out_hbm, didx, ones, wbuf, acc):
        c = lax.axis_index("c")
        s = lax.axis_index("s")

        def fill_ones(k, _):
            ones[pl.ds(k * L, L)] = jnp.ones((L,), jnp.float32)
            return 0

        lax.fori_loop(0, CH // L, fill_ones, 0)

        def fill_zeros(k, _):
            wbuf[pl.ds(k * L, L)] = jnp.zeros((L,), jnp.float32)
            return 0

        lax.fori_loop(0, npt // L, fill_zeros, 0)
        pltpu.sync_copy(wbuf, acc.at[pl.ds(s * npt, npt)])
        plsc.subcore_barrier()

        row0 = c * (epr // NC) + s * cpt
        pltpu.sync_copy(dst_hbm.at[pl.ds(row0, cpt)], didx)

        def step(j, _):
            pltpu.sync_copy(ones, acc.at[didx.at[j]], add=True)
            return 0

        lax.fori_loop(0, cpt, step, 0)
        plsc.subcore_barrier()
        pltpu.sync_copy(acc.at[pl.ds(s * npt, npt)], wbuf)
        pltpu.sync_copy(wbuf, out_hbm.at[pl.ds(c * n_pad + s * npt, npt)])

    mesh = plsc.VectorSubcoreMesh(core_axis_name="c", subcore_axis_name="s",
                                  num_cores=NC, num_subcores=NS)
    return pl.kernel(
        body,
        compiler_params=pltpu.CompilerParams(use_tc_tiling_on_sc=False),
        out_type=jax.ShapeDtypeStruct((NC * n_pad,), jnp.float32),
        mesh=mesh,
        scratch_types=[
            pltpu.VMEM((cpt, CH), jnp.int32),
            pltpu.VMEM((CH,), jnp.float32),
            pltpu.VMEM((npt,), jnp.float32),
            pltpu.VMEM_SHARED((n_pad,), jnp.float32),
        ],
    )(dst_rows)


def _sc_hop(g, src_rows, dst_rows, n_pad):
    epr = src_rows.shape[0]
    cpt_total = epr // NS
    cpt_slow = _round_up(int(cpt_total * 0.30), 8)
    cpt_fast = cpt_total - cpt_slow
    npt = n_pad // NS
    wo = npt // CH

    def body(g_hbm, src_hbm, dst_hbm, out_hbm, sidx, didx, rows0, rows1, acc,
             sem0, sem1):
        c = lax.axis_index("c")
        s = lax.axis_index("s")

        def fill_zero_row(i, _):
            for k in range(F // L):
                rows0[i, pl.ds(k * L, L)] = jnp.zeros((L,), jnp.float32)
            return 0

        lax.fori_loop(0, CH, fill_zero_row, 0)
        for k in range(wo):
            pltpu.sync_copy(rows0, acc.at[pl.ds(s * npt + k * CH, CH)])
        plsc.subcore_barrier()

        def run_chunks(row0, cpt):
            pltpu.sync_copy(src_hbm.at[pl.ds(row0, cpt)], sidx.at[pl.ds(0, cpt)])
            pltpu.sync_copy(dst_hbm.at[pl.ds(row0, cpt)], didx.at[pl.ds(0, cpt)])
            pltpu.async_copy(g_hbm.at[sidx.at[0]], rows0, sem0)
            half = cpt // 2

            def step(i, _):
                j0 = 2 * i
                pltpu.make_async_copy(g_hbm.at[sidx.at[j0]], rows0, sem0).wait()
                pltpu.async_copy(g_hbm.at[sidx.at[j0 + 1]], rows1, sem1)
                pltpu.sync_copy(rows0, acc.at[didx.at[j0]], add=True)
                pltpu.make_async_copy(g_hbm.at[sidx.at[j0 + 1]], rows1, sem1).wait()

                @pl.when(i != half - 1)
                def _():
                    pltpu.async_copy(g_hbm.at[sidx.at[j0 + 2]], rows0, sem0)

                pltpu.sync_copy(rows1, acc.at[didx.at[j0 + 1]], add=True)
                return 0

            lax.fori_loop(0, half, step, 0)

        @pl.when(c == SLOW_CORE)
        def _():
            run_chunks(s * cpt_slow, cpt_slow)

        @pl.when(c != SLOW_CORE)
        def _():
            run_chunks(NS * cpt_slow + s * cpt_fast, cpt_fast)

        plsc.subcore_barrier()
        for k in range(wo):
            sl = pl.ds(s * npt + k * CH, CH)
            pltpu.sync_copy(acc.at[sl], rows0)
            pltpu.sync_copy(rows0, out_hbm.at[c, sl])

    mesh = plsc.VectorSubcoreMesh(core_axis_name="c", subcore_axis_name="s",
                                  num_cores=NC, num_subcores=NS)
    return pl.kernel(
        body,
        compiler_params=pltpu.CompilerParams(use_tc_tiling_on_sc=False),
        out_type=jax.ShapeDtypeStruct((NC, n_pad, F), jnp.float32),
        mesh=mesh,
        scratch_types=[
            pltpu.VMEM((cpt_fast, CH), jnp.int32),
            pltpu.VMEM((cpt_fast, CH), jnp.int32),
            pltpu.VMEM((CH, F), jnp.float32),
            pltpu.VMEM((CH, F), jnp.float32),
            pltpu.VMEM_SHARED((n_pad, F), jnp.float32),
            pltpu.SemaphoreType.DMA,
            pltpu.SemaphoreType.DMA,
        ],
    )(g, src_rows, dst_rows)


def _tc_project(xp, wp, deg_parts3):
    n_pad, d_in = xp.shape

    def body(x_ref, w_ref, dp_ref, g0_ref, dis_ref):
        z = jnp.dot(x_ref[...], w_ref[...],
                    preferred_element_type=jnp.float32,
                    precision=lax.Precision.HIGHEST)
        deg = dp_ref[0] + dp_ref[1] + 1.0
        dis = lax.rsqrt(deg)
        dis_ref[...] = dis
        g0_ref[...] = z * dis

    return pl.pallas_call(
        body,
        grid=(n_pad // R,),
        in_specs=[
            pl.BlockSpec((R, d_in), lambda i: (i, 0)),
            pl.BlockSpec((d_in, F), lambda i: (0, 0)),
            pl.BlockSpec((NC, R, 1), lambda i: (0, i, 0)),
        ],
        out_specs=[
            pl.BlockSpec((R, F), lambda i: (i, 0)),
            pl.BlockSpec((R, 1), lambda i: (i, 0)),
        ],
        out_shape=[
            jax.ShapeDtypeStruct((n_pad, F), jnp.float32),
            jax.ShapeDtypeStruct((n_pad, 1), jnp.float32),
        ],
    )(xp, wp, deg_parts3)


def _tc_rescale(parts, g, dis):
    n_pad = g.shape[0]

    def body(p_ref, g_ref, d_ref, o_ref):
        d = d_ref[...]
        o_ref[...] = (p_ref[0] + p_ref[1] + g_ref[...]) * (d * d)

    return pl.pallas_call(
        body,
        grid=(n_pad // R,),
        in_specs=[
            pl.BlockSpec((NC, R, F), lambda i: (0, i, 0)),
            pl.BlockSpec((R, F), lambda i: (i, 0)),
            pl.BlockSpec((R, 1), lambda i: (i, 0)),
        ],
        out_specs=pl.BlockSpec((R, F), lambda i: (i, 0)),
        out_shape=jax.ShapeDtypeStruct((n_pad, F), jnp.float32),
    )(parts, g, dis)


def _tc_finish(parts, g, dis, b2):
    n_pad = g.shape[0]
    c_out = b2.shape[1]

    def body(p_ref, g_ref, d_ref, b_ref, o_ref):
        h2 = (p_ref[0] + p_ref[1] + g_ref[...]) * d_ref[...]
        logits = h2[:, :c_out] + b_ref[...]
        m = jnp.max(logits, axis=1, keepdims=True)
        lse = m + jnp.log(jnp.sum(jnp.exp(logits - m), axis=1, keepdims=True))
        o_ref[...] = logits - lse

    return pl.pallas_call(
        body,
        grid=(n_pad // R,),
        in_specs=[
            pl.BlockSpec((NC, R, F), lambda i: (0, i, 0)),
            pl.BlockSpec((R, F), lambda i: (i, 0)),
            pl.BlockSpec((R, 1), lambda i: (i, 0)),
            pl.BlockSpec((1, c_out), lambda i: (0, 0)),
        ],
        out_specs=pl.BlockSpec((R, c_out), lambda i: (i, 0)),
        out_shape=jax.ShapeDtypeStruct((n_pad, c_out), jnp.float32),
    )(parts, g, dis, b2)


def kernel(x, edge_index, W, b):
    n, _ = x.shape
    e = edge_index.shape[1]
    n_pad = _round_up(n, NS * CH)
    e_pad = _round_up(e, NC * NS * CH * 8)

    src = jnp.pad(edge_index[0], (0, e_pad - e))
    dst = jnp.pad(edge_index[1], (0, e_pad - e), constant_values=n_pad - 1)
    src_rows = src.reshape(-1, CH)
    dst_rows = dst.reshape(-1, CH)
    xp = jnp.pad(x, ((0, n_pad - n), (0, 0)))
    wp = jnp.pad(W, ((0, 0), (0, F - W.shape[1])))

    deg_parts = _sc_degree(dst_rows, n_pad)
    g0, dis = _tc_project(xp, wp, deg_parts.reshape(NC, n_pad, 1))
    p1 = _sc_hop(g0, src_rows, dst_rows, n_pad)
    g1 = _tc_rescale(p1, g0, dis)
    p2 = _sc_hop(g1, src_rows, dst_rows, n_pad)
    out = _tc_finish(p2, g1, dis, b.reshape(1, -1))
    return out[:n]


# speedup vs baseline: 2.0592x; 2.0592x over previous
import jax
import jax.numpy as jnp
from jax import lax
from jax.experimental import pallas as pl
from jax.experimental.pallas import tpu as pltpu
from jax.experimental.pallas import tpu_sc as plsc

NC, NS, L = 2, 16, 16
SLOW_CORE = 1
F = 48
CH = 128
R = 512


def _round_up(a, m):
    return (a + m - 1) // m * m


def _sc_degree(dst_rows, n_pad):
    epr = dst_rows.shape[0]
    cpt = epr // (NC * NS)
    npt = n_pad // NS

    def body(dst_hbm, out_hbm, didx, ones, wbuf, acc):
        c = lax.axis_index("c")
        s = lax.axis_index("s")

        def fill_ones(k, _):
            ones[pl.ds(k * L, L)] = jnp.ones((L,), jnp.float32)
            return 0

        lax.fori_loop(0, CH // L, fill_ones, 0)

        def fill_zeros(k, _):
            wbuf[pl.ds(k * L, L)] = jnp.zeros((L,), jnp.float32)
            return 0

        lax.fori_loop(0, npt // L, fill_zeros, 0)
        pltpu.sync_copy(wbuf, acc.at[pl.ds(s * npt, npt)])
        plsc.subcore_barrier()

        row0 = c * (epr // NC) + s * cpt
        pltpu.sync_copy(dst_hbm.at[pl.ds(row0, cpt)], didx)

        def step(j, _):
            pltpu.sync_copy(ones, acc.at[didx.at[j]], add=True)
            return 0

        lax.fori_loop(0, cpt, step, 0)
        plsc.subcore_barrier()
        pltpu.sync_copy(acc.at[pl.ds(s * npt, npt)], wbuf)
        pltpu.sync_copy(wbuf, out_hbm.at[pl.ds(c * n_pad + s * npt, npt)])

    mesh = plsc.VectorSubcoreMesh(core_axis_name="c", subcore_axis_name="s",
                                  num_cores=NC, num_subcores=NS)
    return pl.kernel(
        body,
        compiler_params=pltpu.CompilerParams(use_tc_tiling_on_sc=False),
        out_type=jax.ShapeDtypeStruct((NC * n_pad,), jnp.float32),
        mesh=mesh,
        scratch_types=[
            pltpu.VMEM((cpt, CH), jnp.int32),
            pltpu.VMEM((CH,), jnp.float32),
            pltpu.VMEM((npt,), jnp.float32),
            pltpu.VMEM_SHARED((n_pad,), jnp.float32),
        ],
    )(dst_rows)


def _sc_hop(g, src_rows, dst_rows, n_pad):
    epr = src_rows.shape[0]
    cpt_total = epr // NS
    cpt_slow = cpt_total // 2
    cpt_fast = cpt_total - cpt_slow
    npt = n_pad // NS
    wo = npt // CH

    def body(g_hbm, src_hbm, dst_hbm, out_hbm, sidx, didx, rows0, rows1, acc,
             gsp, sem0, sem1):
        c = lax.axis_index("c")
        s = lax.axis_index("s")

        def fill_zero_row(i, _):
            for k in range(F // L):
                rows0[i, pl.ds(k * L, L)] = jnp.zeros((L,), jnp.float32)
            return 0

        lax.fori_loop(0, CH, fill_zero_row, 0)
        for k in range(wo):
            sl = pl.ds(s * npt + k * CH, CH)
            pltpu.sync_copy(rows0, acc.at[sl])
            pltpu.sync_copy(g_hbm.at[sl], rows1)
            pltpu.sync_copy(rows1, gsp.at[sl])
        plsc.subcore_barrier()

        def run_chunks(row0, cpt):
            pltpu.sync_copy(src_hbm.at[pl.ds(row0, cpt)], sidx.at[pl.ds(0, cpt)])
            pltpu.sync_copy(dst_hbm.at[pl.ds(row0, cpt)], didx.at[pl.ds(0, cpt)])
            pltpu.async_copy(gsp.at[sidx.at[0]], rows0, sem0)
            half = cpt // 2

            def step(i, _):
                j0 = 2 * i
                pltpu.make_async_copy(gsp.at[sidx.at[j0]], rows0, sem0).wait()
                pltpu.async_copy(gsp.at[sidx.at[j0 + 1]], rows1, sem1)
                pltpu.sync_copy(rows0, acc.at[didx.at[j0]], add=True)
                pltpu.make_async_copy(gsp.at[sidx.at[j0 + 1]], rows1, sem1).wait()

                @pl.when(i != half - 1)
                def _():
                    pltpu.async_copy(gsp.at[sidx.at[j0 + 2]], rows0, sem0)

                pltpu.sync_copy(rows1, acc.at[didx.at[j0 + 1]], add=True)
                return 0

            lax.fori_loop(0, half, step, 0)

        @pl.when(c == SLOW_CORE)
        def _():
            run_chunks(s * cpt_slow, cpt_slow)

        @pl.when(c != SLOW_CORE)
        def _():
            run_chunks(NS * cpt_slow + s * cpt_fast, cpt_fast)

        plsc.subcore_barrier()
        for k in range(wo):
            sl = pl.ds(s * npt + k * CH, CH)
            pltpu.sync_copy(acc.at[sl], rows0)
            pltpu.sync_copy(rows0, out_hbm.at[c, sl])

    mesh = plsc.VectorSubcoreMesh(core_axis_name="c", subcore_axis_name="s",
                                  num_cores=NC, num_subcores=NS)
    return pl.kernel(
        body,
        compiler_params=pltpu.CompilerParams(use_tc_tiling_on_sc=False),
        out_type=jax.ShapeDtypeStruct((NC, n_pad, F), jnp.float32),
        mesh=mesh,
        scratch_types=[
            pltpu.VMEM((cpt_fast, CH), jnp.int32),
            pltpu.VMEM((cpt_fast, CH), jnp.int32),
            pltpu.VMEM((CH, F), jnp.float32),
            pltpu.VMEM((CH, F), jnp.float32),
            pltpu.VMEM_SHARED((n_pad, F), jnp.float32),
            pltpu.VMEM_SHARED((n_pad, F), jnp.float32),
            pltpu.SemaphoreType.DMA,
            pltpu.SemaphoreType.DMA,
        ],
    )(g, src_rows, dst_rows)


def _tc_project(xp, wp, deg_parts3):
    n_pad, d_in = xp.shape

    def body(x_ref, w_ref, dp_ref, g0_ref, dis_ref):
        z = jnp.dot(x_ref[...], w_ref[...],
                    preferred_element_type=jnp.float32,
                    precision=lax.Precision.HIGHEST)
        deg = dp_ref[0] + dp_ref[1] + 1.0
        dis = lax.rsqrt(deg)
        dis_ref[...] = dis
        g0_ref[...] = z * dis

    return pl.pallas_call(
        body,
        grid=(n_pad // R,),
        in_specs=[
            pl.BlockSpec((R, d_in), lambda i: (i, 0)),
            pl.BlockSpec((d_in, F), lambda i: (0, 0)),
            pl.BlockSpec((NC, R, 1), lambda i: (0, i, 0)),
        ],
        out_specs=[
            pl.BlockSpec((R, F), lambda i: (i, 0)),
            pl.BlockSpec((R, 1), lambda i: (i, 0)),
        ],
        out_shape=[
            jax.ShapeDtypeStruct((n_pad, F), jnp.float32),
            jax.ShapeDtypeStruct((n_pad, 1), jnp.float32),
        ],
    )(xp, wp, deg_parts3)


def _tc_rescale(parts, g, dis):
    n_pad = g.shape[0]

    def body(p_ref, g_ref, d_ref, o_ref):
        d = d_ref[...]
        o_ref[...] = (p_ref[0] + p_ref[1] + g_ref[...]) * (d * d)

    return pl.pallas_call(
        body,
        grid=(n_pad // R,),
        in_specs=[
            pl.BlockSpec((NC, R, F), lambda i: (0, i, 0)),
            pl.BlockSpec((R, F), lambda i: (i, 0)),
            pl.BlockSpec((R, 1), lambda i: (i, 0)),
        ],
        out_specs=pl.BlockSpec((R, F), lambda i: (i, 0)),
        out_shape=jax.ShapeDtypeStruct((n_pad, F), jnp.float32),
    )(parts, g, dis)


def _tc_finish(parts, g, dis, b2):
    n_pad = g.shape[0]
    c_out = b2.shape[1]

    def body(p_ref, g_ref, d_ref, b_ref, o_ref):
        h2 = (p_ref[0] + p_ref[1] + g_ref[...]) * d_ref[...]
        logits = h2[:, :c_out] + b_ref[...]
        m = jnp.max(logits, axis=1, keepdims=True)
        lse = m + jnp.log(jnp.sum(jnp.exp(logits - m), axis=1, keepdims=True))
        o_ref[...] = logits - lse

    return pl.pallas_call(
        body,
        grid=(n_pad // R,),
        in_specs=[
            pl.BlockSpec((NC, R, F), lambda i: (0, i, 0)),
            pl.BlockSpec((R, F), lambda i: (i, 0)),
            pl.BlockSpec((R, 1), lambda i: (i, 0)),
            pl.BlockSpec((1, c_out), lambda i: (0, 0)),
        ],
        out_specs=pl.BlockSpec((R, c_out), lambda i: (i, 0)),
        out_shape=jax.ShapeDtypeStruct((n_pad, c_out), jnp.float32),
    )(parts, g, dis, b2)


def kernel(x, edge_index, W, b):
    n, _ = x.shape
    e = edge_index.shape[1]
    n_pad = _round_up(n, NS * CH)
    e_pad = _round_up(e, NC * NS * CH * 8)

    src = jnp.pad(edge_index[0], (0, e_pad - e))
    dst = jnp.pad(edge_index[1], (0, e_pad - e), constant_values=n_pad - 1)
    src_rows = src.reshape(-1, CH)
    dst_rows = dst.reshape(-1, CH)
    xp = jnp.pad(x, ((0, n_pad - n), (0, 0)))
    wp = jnp.pad(W, ((0, 0), (0, F - W.shape[1])))

    deg_parts = _sc_degree(dst_rows, n_pad)
    g0, dis = _tc_project(xp, wp, deg_parts.reshape(NC, n_pad, 1))
    p1 = _sc_hop(g0, src_rows, dst_rows, n_pad)
    g1 = _tc_rescale(p1, g0, dis)
    p2 = _sc_hop(g1, src_rows, dst_rows, n_pad)
    out = _tc_finish(p2, g1, dis, b.reshape(1, -1))
    return out[:n]


# speedup vs baseline: 2.0860x; 1.0130x over previous
import jax
import jax.numpy as jnp
from jax import lax
from jax.experimental import pallas as pl
from jax.experimental.pallas import tpu as pltpu
from jax.experimental.pallas import tpu_sc as plsc

NC, NS, L = 2, 16, 16
SLOW_CORE = 1
F = 48
CH = 128
R = 512


def _round_up(a, m):
    return (a + m - 1) // m * m


def _sc_degree(dst_rows, n_pad):
    epr = dst_rows.shape[0]
    cpt = epr // (NC * NS)
    npt = n_pad // NS

    def body(dst_hbm, out_hbm, didx, ones, wbuf, acc):
        c = lax.axis_index("c")
        s = lax.axis_index("s")

        def fill_ones(k, _):
            ones[pl.ds(k * L, L)] = jnp.ones((L,), jnp.float32)
            return 0

        lax.fori_loop(0, CH // L, fill_ones, 0)

        def fill_zeros(k, _):
            wbuf[pl.ds(k * L, L)] = jnp.zeros((L,), jnp.float32)
            return 0

        lax.fori_loop(0, npt // L, fill_zeros, 0)
        pltpu.sync_copy(wbuf, acc.at[pl.ds(s * npt, npt)])
        plsc.subcore_barrier()

        row0 = c * (epr // NC) + s * cpt
        pltpu.sync_copy(dst_hbm.at[pl.ds(row0, cpt)], didx)

        def step(j, _):
            pltpu.sync_copy(ones, acc.at[didx.at[j]], add=True)
            return 0

        lax.fori_loop(0, cpt, step, 0)
        plsc.subcore_barrier()
        pltpu.sync_copy(acc.at[pl.ds(s * npt, npt)], wbuf)
        pltpu.sync_copy(wbuf, out_hbm.at[pl.ds(c * n_pad + s * npt, npt)])

    mesh = plsc.VectorSubcoreMesh(core_axis_name="c", subcore_axis_name="s",
                                  num_cores=NC, num_subcores=NS)
    return pl.kernel(
        body,
        compiler_params=pltpu.CompilerParams(use_tc_tiling_on_sc=False),
        out_type=jax.ShapeDtypeStruct((NC * n_pad,), jnp.float32),
        mesh=mesh,
        scratch_types=[
            pltpu.VMEM((cpt, CH), jnp.int32),
            pltpu.VMEM((CH,), jnp.float32),
            pltpu.VMEM((npt,), jnp.float32),
            pltpu.VMEM_SHARED((n_pad,), jnp.float32),
        ],
    )(dst_rows)


def _sc_hop(g, src_rows, dst_rows, n_pad):
    epr = src_rows.shape[0]
    cpt_total = epr // NS
    cpt_slow = cpt_total // 2
    cpt_fast = cpt_total - cpt_slow
    npt = n_pad // NS
    wo = npt // CH

    def body(g_hbm, src_hbm, dst_hbm, out_hbm, sidx, didx, rows0, rows1, acc,
             gsp, sem0, sem1, sems0, sems1):
        c = lax.axis_index("c")
        s = lax.axis_index("s")

        def fill_zero_row(i, _):
            for k in range(F // L):
                rows0[i, pl.ds(k * L, L)] = jnp.zeros((L,), jnp.float32)
            return 0

        lax.fori_loop(0, CH, fill_zero_row, 0)
        for k in range(wo):
            sl = pl.ds(s * npt + k * CH, CH)
            pltpu.sync_copy(rows0, acc.at[sl])
            pltpu.sync_copy(g_hbm.at[sl], rows1)
            pltpu.sync_copy(rows1, gsp.at[sl])
        plsc.subcore_barrier()

        def run_chunks(row0, cpt):
            pltpu.sync_copy(src_hbm.at[pl.ds(row0, cpt)], sidx.at[pl.ds(0, cpt)])
            pltpu.sync_copy(dst_hbm.at[pl.ds(row0, cpt)], didx.at[pl.ds(0, cpt)])
            pltpu.async_copy(gsp.at[sidx.at[0]], rows0, sem0)
            half = cpt // 2

            def step(i, _):
                j0 = 2 * i

                @pl.when(i != 0)
                def _():
                    pltpu.make_async_copy(rows1, acc.at[didx.at[j0]], sems1).wait()

                pltpu.async_copy(gsp.at[sidx.at[j0 + 1]], rows1, sem1)
                pltpu.make_async_copy(gsp.at[sidx.at[j0]], rows0, sem0).wait()
                pltpu.async_copy(rows0, acc.at[didx.at[j0]], sems0, add=True)

                @pl.when(i != half - 1)
                def _():
                    pltpu.make_async_copy(rows0, acc.at[didx.at[j0]], sems0).wait()
                    pltpu.async_copy(gsp.at[sidx.at[j0 + 2]], rows0, sem0)

                pltpu.make_async_copy(gsp.at[sidx.at[j0 + 1]], rows1, sem1).wait()
                pltpu.async_copy(rows1, acc.at[didx.at[j0 + 1]], sems1, add=True)
                return 0

            lax.fori_loop(0, half, step, 0)
            pltpu.make_async_copy(rows0, acc.at[didx.at[0]], sems0).wait()
            pltpu.make_async_copy(rows1, acc.at[didx.at[0]], sems1).wait()

        @pl.when(c == SLOW_CORE)
        def _():
            run_chunks(s * cpt_slow, cpt_slow)

        @pl.when(c != SLOW_CORE)
        def _():
            run_chunks(NS * cpt_slow + s * cpt_fast, cpt_fast)

        plsc.subcore_barrier()
        for k in range(wo):
            sl = pl.ds(s * npt + k * CH, CH)
            pltpu.sync_copy(acc.at[sl], rows0)
            pltpu.sync_copy(rows0, out_hbm.at[c, sl])

    mesh = plsc.VectorSubcoreMesh(core_axis_name="c", subcore_axis_name="s",
                                  num_cores=NC, num_subcores=NS)
    return pl.kernel(
        body,
        compiler_params=pltpu.CompilerParams(use_tc_tiling_on_sc=False),
        out_type=jax.ShapeDtypeStruct((NC, n_pad, F), jnp.float32),
        mesh=mesh,
        scratch_types=[
            pltpu.VMEM((cpt_fast, CH), jnp.int32),
            pltpu.VMEM((cpt_fast, CH), jnp.int32),
            pltpu.VMEM((CH, F), jnp.float32),
            pltpu.VMEM((CH, F), jnp.float32),
            pltpu.VMEM_SHARED((n_pad, F), jnp.float32),
            pltpu.VMEM_SHARED((n_pad, F), jnp.float32),
            pltpu.SemaphoreType.DMA,
            pltpu.SemaphoreType.DMA,
            pltpu.SemaphoreType.DMA,
            pltpu.SemaphoreType.DMA,
        ],
    )(g, src_rows, dst_rows)


def _tc_project(xp, wp, deg_parts3):
    n_pad, d_in = xp.shape

    def body(x_ref, w_ref, dp_ref, g0_ref, dis_ref):
        z = jnp.dot(x_ref[...], w_ref[...],
                    preferred_element_type=jnp.float32,
                    precision=lax.Precision.HIGHEST)
        deg = dp_ref[0] + dp_ref[1] + 1.0
        dis = lax.rsqrt(deg)
        dis_ref[...] = dis
        g0_ref[...] = z * dis

    return pl.pallas_call(
        body,
        grid=(n_pad // R,),
        in_specs=[
            pl.BlockSpec((R, d_in), lambda i: (i, 0)),
            pl.BlockSpec((d_in, F), lambda i: (0, 0)),
            pl.BlockSpec((NC, R, 1), lambda i: (0, i, 0)),
        ],
        out_specs=[
            pl.BlockSpec((R, F), lambda i: (i, 0)),
            pl.BlockSpec((R, 1), lambda i: (i, 0)),
        ],
        out_shape=[
            jax.ShapeDtypeStruct((n_pad, F), jnp.float32),
            jax.ShapeDtypeStruct((n_pad, 1), jnp.float32),
        ],
    )(xp, wp, deg_parts3)


def _tc_rescale(parts, g, dis):
    n_pad = g.shape[0]

    def body(p_ref, g_ref, d_ref, o_ref):
        d = d_ref[...]
        o_ref[...] = (p_ref[0] + p_ref[1] + g_ref[...]) * (d * d)

    return pl.pallas_call(
        body,
        grid=(n_pad // R,),
        in_specs=[
            pl.BlockSpec((NC, R, F), lambda i: (0, i, 0)),
            pl.BlockSpec((R, F), lambda i: (i, 0)),
            pl.BlockSpec((R, 1), lambda i: (i, 0)),
        ],
        out_specs=pl.BlockSpec((R, F), lambda i: (i, 0)),
        out_shape=jax.ShapeDtypeStruct((n_pad, F), jnp.float32),
    )(parts, g, dis)


def _tc_finish(parts, g, dis, b2):
    n_pad = g.shape[0]
    c_out = b2.shape[1]

    def body(p_ref, g_ref, d_ref, b_ref, o_ref):
        h2 = (p_ref[0] + p_ref[1] + g_ref[...]) * d_ref[...]
        logits = h2[:, :c_out] + b_ref[...]
        m = jnp.max(logits, axis=1, keepdims=True)
        lse = m + jnp.log(jnp.sum(jnp.exp(logits - m), axis=1, keepdims=True))
        o_ref[...] = logits - lse

    return pl.pallas_call(
        body,
        grid=(n_pad // R,),
        in_specs=[
            pl.BlockSpec((NC, R, F), lambda i: (0, i, 0)),
            pl.BlockSpec((R, F), lambda i: (i, 0)),
            pl.BlockSpec((R, 1), lambda i: (i, 0)),
            pl.BlockSpec((1, c_out), lambda i: (0, 0)),
        ],
        out_specs=pl.BlockSpec((R, c_out), lambda i: (i, 0)),
        out_shape=jax.ShapeDtypeStruct((n_pad, c_out), jnp.float32),
    )(parts, g, dis, b2)


def kernel(x, edge_index, W, b):
    n, _ = x.shape
    e = edge_index.shape[1]
    n_pad = _round_up(n, NS * CH)
    e_pad = _round_up(e, NC * NS * CH * 8)

    src = jnp.pad(edge_index[0], (0, e_pad - e))
    dst = jnp.pad(edge_index[1], (0, e_pad - e), constant_values=n_pad - 1)
    src_rows = src.reshape(-1, CH)
    dst_rows = dst.reshape(-1, CH)
    xp = jnp.pad(x, ((0, n_pad - n), (0, 0)))
    wp = jnp.pad(W, ((0, 0), (0, F - W.shape[1])))

    deg_parts = _sc_degree(dst_rows, n_pad)
    g0, dis = _tc_project(xp, wp, deg_parts.reshape(NC, n_pad, 1))
    p1 = _sc_hop(g0, src_rows, dst_rows, n_pad)
    g1 = _tc_rescale(p1, g0, dis)
    p2 = _sc_hop(g1, src_rows, dst_rows, n_pad)
    out = _tc_finish(p2, g1, dis, b.reshape(1, -1))
    return out[:n]


# speedup vs baseline: 2.2183x; 1.0634x over previous
import jax
import jax.numpy as jnp
from jax import lax
from jax.experimental import pallas as pl
from jax.experimental.pallas import tpu as pltpu
from jax.experimental.pallas import tpu_sc as plsc

NC, NS, L = 2, 16, 16
F = 48
CH = 128
R = 512


def _round_up(a, m):
    return (a + m - 1) // m * m


def _sc_degree(ei3, n_pad):
    epr = ei3.shape[1]
    cpt = epr // (NC * NS)
    extra = epr - cpt * NC * NS
    npt = n_pad // NS

    def body(ei_hbm, out_hbm, didx, ones, wbuf, acc):
        c = lax.axis_index("c")
        s = lax.axis_index("s")

        def fill_ones(k, _):
            ones[pl.ds(k * L, L)] = jnp.ones((L,), jnp.float32)
            return 0

        lax.fori_loop(0, CH // L, fill_ones, 0)

        def fill_zeros(k, _):
            wbuf[pl.ds(k * L, L)] = jnp.zeros((L,), jnp.float32)
            return 0

        lax.fori_loop(0, npt // L, fill_zeros, 0)
        pltpu.sync_copy(wbuf, acc.at[pl.ds(s * npt, npt)])
        plsc.subcore_barrier()

        row0 = (c * NS + s) * cpt
        pltpu.sync_copy(ei_hbm.at[1, pl.ds(row0, cpt)], didx)

        def step(j, _):
            pltpu.sync_copy(ones, acc.at[didx.at[j]], add=True)
            return 0

        lax.fori_loop(0, cpt, step, 0)

        @pl.when(jnp.logical_and(c == 0, s < extra))
        def _():
            pltpu.sync_copy(ei_hbm.at[1, pl.ds(NC * NS * cpt + s, 1)],
                            didx.at[pl.ds(0, 1)])
            pltpu.sync_copy(ones, acc.at[didx.at[0]], add=True)

        plsc.subcore_barrier()
        pltpu.sync_copy(acc.at[pl.ds(s * npt, npt)], wbuf)
        pltpu.sync_copy(wbuf, out_hbm.at[pl.ds(c * n_pad + s * npt, npt)])

    mesh = plsc.VectorSubcoreMesh(core_axis_name="c", subcore_axis_name="s",
                                  num_cores=NC, num_subcores=NS)
    return pl.kernel(
        body,
        compiler_params=pltpu.CompilerParams(use_tc_tiling_on_sc=False),
        out_type=jax.ShapeDtypeStruct((NC * n_pad,), jnp.float32),
        mesh=mesh,
        scratch_types=[
            pltpu.VMEM((cpt, CH), jnp.int32),
            pltpu.VMEM((CH,), jnp.float32),
            pltpu.VMEM((npt,), jnp.float32),
            pltpu.VMEM_SHARED((n_pad,), jnp.float32),
        ],
    )(ei3)


def _sc_hop(g, ei3, n_pad):
    epr = ei3.shape[1]
    cpt = epr // (NC * NS)
    extra = epr - cpt * NC * NS
    npt = n_pad // NS
    wo = npt // CH

    def body(g_hbm, ei_hbm, out_hbm, sidx, didx, rows0, rows1, acc,
             gsp, sem0, sem1, sems0, sems1):
        c = lax.axis_index("c")
        s = lax.axis_index("s")

        def fill_zero_row(i, _):
            for k in range(F // L):
                rows0[i, pl.ds(k * L, L)] = jnp.zeros((L,), jnp.float32)
            return 0

        lax.fori_loop(0, CH, fill_zero_row, 0)
        for k in range(wo):
            sl = pl.ds(s * npt + k * CH, CH)
            pltpu.sync_copy(rows0, acc.at[sl])
            pltpu.sync_copy(g_hbm.at[sl], rows1)
            pltpu.sync_copy(rows1, gsp.at[sl])
        plsc.subcore_barrier()

        def run_chunks(row0):
            pltpu.sync_copy(ei_hbm.at[0, pl.ds(row0, cpt)], sidx)
            pltpu.sync_copy(ei_hbm.at[1, pl.ds(row0, cpt)], didx)
            pltpu.async_copy(gsp.at[sidx.at[0]], rows0, sem0)
            half = cpt // 2

            def step(i, _):
                j0 = 2 * i

                @pl.when(i != 0)
                def _():
                    pltpu.make_async_copy(rows1, acc.at[didx.at[j0]], sems1).wait()

                pltpu.async_copy(gsp.at[sidx.at[j0 + 1]], rows1, sem1)
                pltpu.make_async_copy(gsp.at[sidx.at[j0]], rows0, sem0).wait()
                pltpu.async_copy(rows0, acc.at[didx.at[j0]], sems0, add=True)

                @pl.when(i != half - 1)
                def _():
                    pltpu.make_async_copy(rows0, acc.at[didx.at[j0]], sems0).wait()
                    pltpu.async_copy(gsp.at[sidx.at[j0 + 2]], rows0, sem0)

                pltpu.make_async_copy(gsp.at[sidx.at[j0 + 1]], rows1, sem1).wait()
                pltpu.async_copy(rows1, acc.at[didx.at[j0 + 1]], sems1, add=True)
                return 0

            lax.fori_loop(0, half, step, 0)
            pltpu.make_async_copy(rows0, acc.at[didx.at[0]], sems0).wait()
            pltpu.make_async_copy(rows1, acc.at[didx.at[0]], sems1).wait()

        run_chunks((c * NS + s) * cpt)

        @pl.when(jnp.logical_and(c == 0, s < extra))
        def _():
            pltpu.sync_copy(ei_hbm.at[0, pl.ds(NC * NS * cpt + s, 1)],
                            sidx.at[pl.ds(0, 1)])
            pltpu.sync_copy(ei_hbm.at[1, pl.ds(NC * NS * cpt + s, 1)],
                            didx.at[pl.ds(0, 1)])
            pltpu.sync_copy(gsp.at[sidx.at[0]], rows0)
            pltpu.sync_copy(rows0, acc.at[didx.at[0]], add=True)

        plsc.subcore_barrier()
        for k in range(wo):
            sl = pl.ds(s * npt + k * CH, CH)
            pltpu.sync_copy(acc.at[sl], rows0)
            pltpu.sync_copy(rows0, out_hbm.at[c, sl])

    mesh = plsc.VectorSubcoreMesh(core_axis_name="c", subcore_axis_name="s",
                                  num_cores=NC, num_subcores=NS)
    return pl.kernel(
        body,
        compiler_params=pltpu.CompilerParams(use_tc_tiling_on_sc=False),
        out_type=jax.ShapeDtypeStruct((NC, n_pad, F), jnp.float32),
        mesh=mesh,
        scratch_types=[
            pltpu.VMEM((cpt, CH), jnp.int32),
            pltpu.VMEM((cpt, CH), jnp.int32),
            pltpu.VMEM((CH, F), jnp.float32),
            pltpu.VMEM((CH, F), jnp.float32),
            pltpu.VMEM_SHARED((n_pad, F), jnp.float32),
            pltpu.VMEM_SHARED((n_pad, F), jnp.float32),
            pltpu.SemaphoreType.DMA,
            pltpu.SemaphoreType.DMA,
            pltpu.SemaphoreType.DMA,
            pltpu.SemaphoreType.DMA,
        ],
    )(g, ei3)


def _tc_project(xp, wp, deg_parts3):
    n_pad, d_in = xp.shape

    def body(x_ref, w_ref, dp_ref, g0_ref, dis_ref):
        z = jnp.dot(x_ref[...], w_ref[...],
                    preferred_element_type=jnp.float32,
                    precision=lax.Precision.HIGHEST)
        deg = dp_ref[0] + dp_ref[1] + 1.0
        dis = lax.rsqrt(deg)
        dis_ref[...] = dis
        g0_ref[...] = z * dis

    return pl.pallas_call(
        body,
        grid=(n_pad // R,),
        in_specs=[
            pl.BlockSpec((R, d_in), lambda i: (i, 0)),
            pl.BlockSpec((d_in, F), lambda i: (0, 0)),
            pl.BlockSpec((NC, R, 1), lambda i: (0, i, 0)),
        ],
        out_specs=[
            pl.BlockSpec((R, F), lambda i: (i, 0)),
            pl.BlockSpec((R, 1), lambda i: (i, 0)),
        ],
        out_shape=[
            jax.ShapeDtypeStruct((n_pad, F), jnp.float32),
            jax.ShapeDtypeStruct((n_pad, 1), jnp.float32),
        ],
    )(xp, wp, deg_parts3)


def _tc_rescale(parts, g, dis):
    n_pad = g.shape[0]

    def body(p_ref, g_ref, d_ref, o_ref):
        d = d_ref[...]
        o_ref[...] = (p_ref[0] + p_ref[1] + g_ref[...]) * (d * d)

    return pl.pallas_call(
        body,
        grid=(n_pad // R,),
        in_specs=[
            pl.BlockSpec((NC, R, F), lambda i: (0, i, 0)),
            pl.BlockSpec((R, F), lambda i: (i, 0)),
            pl.BlockSpec((R, 1), lambda i: (i, 0)),
        ],
        out_specs=pl.BlockSpec((R, F), lambda i: (i, 0)),
        out_shape=jax.ShapeDtypeStruct((n_pad, F), jnp.float32),
    )(parts, g, dis)


def _tc_finish(parts, g, dis, b2):
    n_pad = g.shape[0]
    c_out = b2.shape[1]

    def body(p_ref, g_ref, d_ref, b_ref, o_ref):
        h2 = (p_ref[0] + p_ref[1] + g_ref[...]) * d_ref[...]
        logits = h2[:, :c_out] + b_ref[...]
        m = jnp.max(logits, axis=1, keepdims=True)
        lse = m + jnp.log(jnp.sum(jnp.exp(logits - m), axis=1, keepdims=True))
        o_ref[...] = logits - lse

    return pl.pallas_call(
        body,
        grid=(n_pad // R,),
        in_specs=[
            pl.BlockSpec((NC, R, F), lambda i: (0, i, 0)),
            pl.BlockSpec((R, F), lambda i: (i, 0)),
            pl.BlockSpec((R, 1), lambda i: (i, 0)),
            pl.BlockSpec((1, c_out), lambda i: (0, 0)),
        ],
        out_specs=pl.BlockSpec((R, c_out), lambda i: (i, 0)),
        out_shape=jax.ShapeDtypeStruct((n_pad, c_out), jnp.float32),
    )(parts, g, dis, b2)


def kernel(x, edge_index, W, b):
    n, _ = x.shape
    e = edge_index.shape[1]
    n_pad = _round_up(n, NS * CH)

    ei3 = edge_index.reshape(2, e // CH, CH)
    xp = jnp.pad(x, ((0, n_pad - n), (0, 0)))
    wp = jnp.pad(W, ((0, 0), (0, F - W.shape[1])))

    deg_parts = _sc_degree(ei3, n_pad)
    g0, dis = _tc_project(xp, wp, deg_parts.reshape(NC, n_pad, 1))
    p1 = _sc_hop(g0, ei3, n_pad)
    g1 = _tc_rescale(p1, g0, dis)
    p2 = _sc_hop(g1, ei3, n_pad)
    out = _tc_finish(p2, g1, dis, b.reshape(1, -1))
    return out[:n]
